# baseline (speedup 1.0000x reference)
import functools

import jax
import jax.numpy as jnp
from jax import lax
from jax.experimental import pallas as pl
from jax.experimental.pallas import tpu as pltpu
from jax.experimental.pallas import tpu_sc as plsc

H = 768
S = 77
SP = 80
B = 4096
NC, NS = 2, 16
SEQ_PER_SUB = B // NS
LANES = 16
PIECES = H // 128
PPS = SP * PIECES
PPW = PPS // NC
CH = 32
PPC = CH * PPW

_mesh = plsc.VectorSubcoreMesh(core_axis_name="c", subcore_axis_name="s")


@functools.partial(
    pl.kernel,
    out_type=jax.ShapeDtypeStruct((B, PPS, 128), jnp.float32),
    mesh=_mesh,
    scratch_types=[
        pltpu.VMEM((PPC,), jnp.int32),
        pltpu.VMEM((PPW, 128), jnp.float32),
        pltpu.VMEM((PPW, 128), jnp.float32),
        pltpu.VMEM((PPW, 128), jnp.float32),
        pltpu.SemaphoreType.DMA,
        pltpu.SemaphoreType.DMA,
        pltpu.SemaphoreType.DMA,
        pltpu.SemaphoreType.DMA,
    ],
)
def _embed(idsw_hbm, tab_hbm, pos_hbm, out_hbm,
           idx_v, pos_v, buf0, buf1, g0, g1, so0, so1):
    c = lax.axis_index("c")
    sid = lax.axis_index("s")
    seq0 = sid * SEQ_PER_SUB
    woff = pl.multiple_of((sid * NC + c) * (SEQ_PER_SUB * PPW), 8)
    cof = pl.multiple_of(c * PPW, 8)

    pltpu.sync_copy(pos_hbm.at[pl.ds(cof, PPW)], pos_v)

    bufs = (buf0, buf1)
    gsem = (g0, g1)
    ssem = (so0, so1)

    def refresh(j):
        pltpu.sync_copy(
            idsw_hbm.at[pl.ds(woff + (j // CH) * PPC, PPC)], idx_v)

    def gstart(j, b):
        ioff = pl.multiple_of((j % CH) * PPW, 8)
        half = PPW // 2
        pltpu.async_copy(tab_hbm.at[idx_v.at[pl.ds(ioff, half)]],
                         bufs[b].at[pl.ds(0, half)], gsem[b])
        pltpu.async_copy(tab_hbm.at[idx_v.at[pl.ds(ioff + half, half)]],
                         bufs[b].at[pl.ds(half, half)], gsem[b])

    def gwait(b):
        pltpu.make_async_copy(
            tab_hbm.at[pl.ds(0, PPW)], bufs[b], gsem[b]).wait()

    def sstart(j, b):
        pltpu.async_copy(
            bufs[b], out_hbm.at[seq0 + j, pl.ds(cof, PPW)], ssem[b])

    def swait(j, b):
        pltpu.make_async_copy(
            bufs[b], out_hbm.at[seq0 + j, pl.ds(cof, PPW)], ssem[b]).wait()

    def add_pos(b):
        def add_row(r, c2):
            for g in range(128 // LANES):
                sl = pl.ds(g * LANES, LANES)
                plsc.addupdate(bufs[b].at[r, sl], pos_v[r, sl])
            return c2
        lax.fori_loop(0, PPW, add_row, 0)

    refresh(0)
    gstart(0, 0)

    def outer(i2, carry):
        for b in range(2):
            ob = 1 - b
            i = i2 * 2 + b
            gwait(b)

            @pl.when(i >= 1)
            def _():
                swait(i - 1, ob)

            @pl.when(jnp.logical_and((i + 1) % CH == 0,
                                     i + 1 <= SEQ_PER_SUB - 1))
            def _():
                refresh(i + 1)

            @pl.when(i + 1 <= SEQ_PER_SUB - 1)
            def _():
                gstart(i + 1, ob)

            add_pos(b)
            sstart(i, b)
        return carry

    lax.fori_loop(0, SEQ_PER_SUB // 2, outer, 0)
    swait(SEQ_PER_SUB - 1, 1)


def kernel(input_ids, token_embedding, position_embedding):
    ids_pad = jnp.pad(input_ids, ((0, 0), (0, SP - S)))
    ids6 = ids_pad[..., None] * PIECES + jnp.arange(
        PIECES, dtype=ids_pad.dtype)
    ids_w = (ids6.reshape(NS, SEQ_PER_SUB, NC, PPW)
             .transpose(0, 2, 1, 3).reshape(-1))
    tab128 = token_embedding.reshape(-1, 128)
    pos_pad = jnp.pad(position_embedding, ((0, SP - S), (0, 0)))
    pos128 = pos_pad.reshape(-1, 128)
    out = _embed(ids_w, tab128, pos128)
    return out.reshape(B, SP, H)[:, :S, :]
